# baseline (speedup 1.0000x reference)
import jax
import jax.numpy as jnp
from jax.experimental import pallas as pl
from jax.experimental.pallas import tpu as pltpu

_N, _D, _E, _K = 8192, 2048, 16, 2
_EP = 128
_TN = 512
_P = 4
_PC = _D // _P
_NT = _N // _TN


def _moe_body(x_ref, wr_ref, br_ref, we_ref, be_ref, out_ref, loss_ref,
              web_ref, xb_ref, w0_ref, w1_ref):
    s = pl.program_id(0)
    t = jnp.maximum(s - 1, 0)
    p = t % _P

    @pl.when(s < _P)
    def _cast_chunk():
        web_ref[:, pl.ds(jnp.minimum(s, _P - 1) * _PC, _PC), :] = (
            we_ref[...].astype(jnp.bfloat16))

    @pl.when((s >= 1) & (p == 0))
    def _router():
        xb = x_ref[...].astype(jnp.bfloat16)
        xb_ref[...] = xb
        logits = jax.lax.dot_general(
            xb, wr_ref[...], (((1,), (1,)), ((), ())),
            preferred_element_type=jnp.float32)
        logits = logits + br_ref[...]
        m = jnp.max(logits, axis=-1, keepdims=True)
        e = jnp.exp(logits - m)
        su = jnp.sum(e, axis=-1, keepdims=True)
        m1 = jnp.max(e, axis=-1, keepdims=True)
        lane = jax.lax.broadcasted_iota(jnp.int32, (_TN, _EP), 1)
        first_idx = jnp.min(jnp.where(e == m1, lane, _EP), axis=-1,
                            keepdims=True)
        e_masked = jnp.where(lane == first_idx, -jnp.inf, e)
        m2 = jnp.max(e_masked, axis=-1, keepdims=True)
        tot = m1 + m2
        w0_ref[...] = m1 / tot
        w1_ref[...] = m2 / tot
        loss_ref[...] = jnp.sum(tot / su, keepdims=True)[None] * (1.0 / _N)

    @pl.when(s >= 1)
    def _panel():
        xb = xb_ref[...]
        w0 = w0_ref[...]
        w1 = w1_ref[...]
        wrows = pl.ds(p * _PC, _PC)
        a0 = jax.lax.dot_general(
            xb, web_ref[0, wrows, :], (((1,), (1,)), ((), ())),
            preferred_element_type=jnp.float32)
        a1 = jax.lax.dot_general(
            xb, web_ref[1, wrows, :], (((1,), (1,)), ((), ())),
            preferred_element_type=jnp.float32)
        out_ref[...] = (w0 * a0 + w1 * a1
                        + w0 * be_ref[0, p, :][None, :]
                        + w1 * be_ref[1, p, :][None, :])


def kernel(x, Wr, br, We, be):
    wr_p = jnp.zeros((_EP, _D), jnp.bfloat16).at[:_E].set(Wr.astype(jnp.bfloat16))
    br_p = jnp.full((1, _EP), -1e30, jnp.float32).at[0, :_E].set(br)
    be3 = be.reshape(_K, _P, _PC)

    grid = 1 + _NT * _P
    out, loss_parts = pl.pallas_call(
        _moe_body,
        grid=(grid,),
        in_specs=[
            pl.BlockSpec((_TN, _D),
                         lambda s: (jnp.maximum(s - 1, 0) // _P, 0)),
            pl.BlockSpec((_EP, _D), lambda s: (0, 0)),
            pl.BlockSpec((1, _EP), lambda s: (0, 0)),
            pl.BlockSpec((_K, _PC, _D),
                         lambda s: (0, jnp.minimum(s, _P - 1), 0)),
            pl.BlockSpec((_K, _P, _PC), lambda s: (0, 0, 0)),
        ],
        out_specs=[
            pl.BlockSpec((_TN, _PC),
                         lambda s: (jnp.maximum(s - 1, 0) // _P,
                                    jnp.maximum(s - 1, 0) % _P)),
            pl.BlockSpec((1, 1, 1),
                         lambda s: (jnp.maximum(s - 1, 0) // _P, 0, 0)),
        ],
        out_shape=[
            jax.ShapeDtypeStruct((_N, _D), jnp.float32),
            jax.ShapeDtypeStruct((_NT, 1, 1), jnp.float32),
        ],
        scratch_shapes=[
            pltpu.VMEM((_K, _D, _D), jnp.bfloat16),
            pltpu.VMEM((_TN, _D), jnp.bfloat16),
            pltpu.VMEM((_TN, 1), jnp.float32),
            pltpu.VMEM((_TN, 1), jnp.float32),
        ],
    )(x, wr_p, br_p, We, be3)
    return out, jnp.sum(loss_parts)


# speedup vs baseline: 1.2367x; 1.2367x over previous
import jax
import jax.numpy as jnp
from jax.experimental import pallas as pl
from jax.experimental.pallas import tpu as pltpu

_N, _D, _E, _K = 8192, 2048, 16, 2
_EP = 128
_TN = 512
_CAST = 8
_CROWS = _D // _CAST


def _moe_body(x_ref, wr_ref, br_ref, we_ref, be_ref, out_ref, loss_ref,
              web_ref):
    pid = pl.program_id(0)

    @pl.when(pid < _CAST)
    def _cast_phase():
        web_ref[:, pl.ds(pid * _CROWS, _CROWS), :] = (
            we_ref[...].astype(jnp.bfloat16))

    @pl.when(pid == 0)
    def _init_loss():
        loss_ref[...] = jnp.zeros_like(loss_ref)

    @pl.when(pid >= _CAST)
    def _compute_phase():
        xb = x_ref[...].astype(jnp.bfloat16)

        logits = jax.lax.dot_general(
            xb, wr_ref[...], (((1,), (1,)), ((), ())),
            preferred_element_type=jnp.float32)
        logits = logits + br_ref[...]
        m = jnp.max(logits, axis=-1, keepdims=True)
        e = jnp.exp(logits - m)
        s = jnp.sum(e, axis=-1, keepdims=True)
        m1 = jnp.max(e, axis=-1, keepdims=True)
        lane = jax.lax.broadcasted_iota(jnp.int32, (_TN, _EP), 1)
        first_idx = jnp.min(jnp.where(e == m1, lane, _EP), axis=-1,
                            keepdims=True)
        e_masked = jnp.where(lane == first_idx, -jnp.inf, e)
        m2 = jnp.max(e_masked, axis=-1, keepdims=True)
        tot = m1 + m2
        w0 = m1 / tot
        w1 = m2 / tot

        loss_ref[...] += jnp.sum(tot / s, keepdims=True) * (1.0 / _N)

        a0 = jax.lax.dot_general(
            xb, web_ref[0], (((1,), (1,)), ((), ())),
            preferred_element_type=jnp.float32)
        out_ref[...] = w0 * a0 + (w0 * be_ref[0:1, :] + w1 * be_ref[1:2, :])
        a1 = jax.lax.dot_general(
            xb, web_ref[1], (((1,), (1,)), ((), ())),
            preferred_element_type=jnp.float32)
        out_ref[...] += w1 * a1


def kernel(x, Wr, br, We, be):
    wr_p = jnp.zeros((_EP, _D), jnp.bfloat16).at[:_E].set(Wr.astype(jnp.bfloat16))
    br_p = jnp.full((1, _EP), -1e30, jnp.float32).at[0, :_E].set(br)

    grid = _CAST + _N // _TN
    out, loss = pl.pallas_call(
        _moe_body,
        grid=(grid,),
        in_specs=[
            pl.BlockSpec((_TN, _D),
                         lambda n: (jnp.maximum(n - _CAST, 0), 0)),
            pl.BlockSpec((_EP, _D), lambda n: (0, 0)),
            pl.BlockSpec((1, _EP), lambda n: (0, 0)),
            pl.BlockSpec((_K, _CROWS, _D),
                         lambda n: (0, jnp.minimum(n, _CAST - 1), 0)),
            pl.BlockSpec((_K, _D), lambda n: (0, 0)),
        ],
        out_specs=[
            pl.BlockSpec((_TN, _D),
                         lambda n: (jnp.maximum(n - _CAST, 0), 0)),
            pl.BlockSpec((1, 1), lambda n: (0, 0)),
        ],
        out_shape=[
            jax.ShapeDtypeStruct((_N, _D), jnp.float32),
            jax.ShapeDtypeStruct((1, 1), jnp.float32),
        ],
        scratch_shapes=[pltpu.VMEM((_K, _D, _D), jnp.bfloat16)],
    )(x, wr_p, br_p, We, be)
    return out, loss[0, 0]
